# Initial kernel scaffold; baseline (speedup 1.0000x reference)
import jax
import jax.numpy as jnp
from jax.experimental import pallas as pl


def kernel(xyz, features, W1, g1, b1, W2, g2, b2, W3, g3, b3):
    raise NotImplementedError("write your pallas kernel here")


# speedup vs baseline: 12.3352x; 12.3352x over previous
import functools

import jax
import jax.numpy as jnp
from jax import lax
from jax.experimental import pallas as pl
from jax.experimental.pallas import tpu as pltpu
from jax.experimental.pallas import tpu_sc as plsc

B = 8
N = 8192
S = 1024
K = 32
CH0 = 128
C1, C2, C3 = 32, 32, 64
R2 = 0.2 ** 2
EPSV = 1e-5
ROWS = B * S
XROWS = ROWS * K
TS = 128
MTILE = 2048
MGRID = XROWS // MTILE
NTOT = float(XROWS)

_PREC = lax.Precision.DEFAULT


def _fps_body(xyzT_ref, cpad_ref, dist_ref):
    x = xyzT_ref[0]
    y = xyzT_ref[1]
    z = xyzT_ref[2]
    col = lax.broadcasted_iota(jnp.int32, (B, N), 1)
    dist_ref[...] = jnp.full((B, N), jnp.inf, dtype=jnp.float32)

    def step(t, far):
        fm = col == far
        cx = jnp.sum(jnp.where(fm, x, 0.0), axis=1, keepdims=True)
        cy = jnp.sum(jnp.where(fm, y, 0.0), axis=1, keepdims=True)
        cz = jnp.sum(jnp.where(fm, z, 0.0), axis=1, keepdims=True)
        crow = jnp.concatenate(
            [cx[:, :, None], cy[:, :, None], cz[:, :, None],
             jnp.zeros((B, 1, 13), jnp.float32)], axis=2)
        cpad_ref[:, pl.ds(t, 1), :] = crow
        dx = x - cx
        dy = y - cy
        dz = z - cz
        d = (dx * dx + dy * dy) + dz * dz
        dn = jnp.minimum(dist_ref[...], d)
        dist_ref[...] = dn
        m = jnp.max(dn, axis=1, keepdims=True)
        far_new = jnp.min(jnp.where(dn == m, col, N), axis=1, keepdims=True)
        return far_new

    lax.fori_loop(0, S, step, jnp.zeros((B, 1), jnp.int32))


def _run_fps(xyz):
    xyzT = jnp.transpose(xyz, (2, 0, 1))
    return pl.pallas_call(
        _fps_body,
        out_shape=jax.ShapeDtypeStruct((B, S, 16), jnp.float32),
        scratch_shapes=[pltpu.VMEM((B, N), jnp.float32)],
    )(xyzT)


def _bq_body(xyzT_ref, cpad_ref, out_ref, d_ref):
    p3 = xyzT_ref[0]
    cm = cpad_ref[0][:, 0:3]
    px = p3[0:1, :]
    py = p3[1:2, :]
    pz = p3[2:3, :]
    pn2 = (px * px + py * py) + pz * pz
    cn2 = jnp.sum(cm * cm, axis=1, keepdims=True)
    dmm = lax.dot_general(cm, p3, (((1,), (0,)), ((), ())),
                          precision=_PREC)
    sq = (cn2 + pn2) - 2.0 * dmm
    col = lax.broadcasted_iota(jnp.int32, (TS, N), 1)
    rowmin = jnp.min(sq, axis=1, keepdims=True)
    nearest = jnp.min(jnp.where(sq == rowmin, col, N), axis=1, keepdims=True)
    d_ref[...] = jnp.where(sq <= R2, sq, jnp.inf)
    base = pl.program_id(0) * N
    kcol = lax.broadcasted_iota(jnp.int32, (TS, K), 1)

    def step(k, acc):
        d = d_ref[...]
        m = jnp.min(d, axis=1, keepdims=True)
        idx = jnp.min(jnp.where(d == m, col, N), axis=1, keepdims=True)
        sel = jnp.where(m == jnp.inf, nearest, idx)
        d_ref[...] = jnp.where(col == idx, jnp.inf, d)
        return jnp.where(kcol == k, sel, acc)

    acc = lax.fori_loop(0, K, step, jnp.zeros((TS, K), jnp.int32))
    out_ref[0] = acc + base


def _run_ballquery(xyz, cpad):
    xyzT = jnp.transpose(xyz, (0, 2, 1))
    return pl.pallas_call(
        _bq_body,
        grid=(B, S // TS),
        in_specs=[
            pl.BlockSpec((1, 3, N), lambda b, s: (b, 0, 0)),
            pl.BlockSpec((1, TS, 16), lambda b, s: (b, s, 0)),
        ],
        out_specs=pl.BlockSpec((1, TS, K), lambda b, s: (b, s, 0)),
        out_shape=jax.ShapeDtypeStruct((B, S, K), jnp.int32),
        scratch_shapes=[pltpu.VMEM((TS, N), jnp.float32)],
    )(xyzT, cpad)


_NW = 32
_RPW = XROWS // _NW
_GCHUNK = 256


def _sc_gather_body(table_hbm, gidx_hbm, out_hbm, idx_v, rows_v, sem):
    wid = lax.axis_index("s") * 2 + lax.axis_index("c")
    base = wid * _RPW

    def chunk(g, carry):
        off = base + g * _GCHUNK
        pltpu.sync_copy(gidx_hbm.at[pl.ds(off, _GCHUNK)], idx_v)
        pltpu.async_copy(table_hbm.at[idx_v], rows_v, sem).wait()
        pltpu.sync_copy(rows_v, out_hbm.at[pl.ds(off, _GCHUNK)])
        return carry

    lax.fori_loop(0, _RPW // _GCHUNK, chunk, 0)


def _gather_rows(table, gidx_flat):
    mesh = plsc.VectorSubcoreMesh(core_axis_name="c", subcore_axis_name="s")
    fn = functools.partial(
        pl.kernel,
        out_type=jax.ShapeDtypeStruct((XROWS, CH0), jnp.float32),
        mesh=mesh,
        scratch_types=[
            pltpu.VMEM((_GCHUNK,), jnp.int32),
            pltpu.VMEM((_GCHUNK, CH0), jnp.float32),
            pltpu.SemaphoreType.DMA,
        ],
    )(_sc_gather_body)
    return fn(table, gidx_flat)


def _stats(p_ref):
    ps = p_ref[...]
    mu = jnp.sum(ps[:, 0, :], axis=0, keepdims=True) / NTOT
    ex2 = jnp.sum(ps[:, 1, :], axis=0, keepdims=True) / NTOT
    var = ex2 - mu * mu
    inv = lax.rsqrt(var + EPSV)
    return mu, inv


def _x1_of(xc_ref, nx_ref, w1_ref):
    xc = xc_ref[...]
    cb = nx_ref[...]
    cexp = jnp.reshape(
        jnp.broadcast_to(cb[:, None, :], (MTILE // K, K, CH0)), (MTILE, CH0))
    x0 = xc - cexp
    return lax.dot_general(x0, w1_ref[...], (((1,), (1,)), ((), ())),
                           precision=_PREC)


def _norm_relu(x, p_ref, g_ref, b_ref):
    mu, inv = _stats(p_ref)
    return jnp.maximum((x - mu) * inv * g_ref[...] + b_ref[...], 0.0)


def _wr_partials(x, pout_ref):
    pout_ref[0, 0, :] = jnp.sum(x, axis=0)
    pout_ref[0, 1, :] = jnp.sum(x * x, axis=0)


def _t1_body(xc_ref, nx_ref, w1_ref, p1o_ref):
    _wr_partials(_x1_of(xc_ref, nx_ref, w1_ref), p1o_ref)


def _t2_body(xc_ref, nx_ref, w1_ref, w2_ref, g1_ref, b1_ref, p1_ref, p2o_ref):
    h1 = _norm_relu(_x1_of(xc_ref, nx_ref, w1_ref), p1_ref, g1_ref, b1_ref)
    x2 = lax.dot_general(h1, w2_ref[...], (((1,), (1,)), ((), ())),
                         precision=_PREC)
    _wr_partials(x2, p2o_ref)


def _t3_body(xc_ref, nx_ref, w1_ref, w2_ref, w3_ref, g1_ref, b1_ref,
             g2_ref, b2_ref, p1_ref, p2_ref, p3o_ref):
    h1 = _norm_relu(_x1_of(xc_ref, nx_ref, w1_ref), p1_ref, g1_ref, b1_ref)
    x2 = lax.dot_general(h1, w2_ref[...], (((1,), (1,)), ((), ())),
                         precision=_PREC)
    h2 = _norm_relu(x2, p2_ref, g2_ref, b2_ref)
    x3 = lax.dot_general(h2, w3_ref[...], (((1,), (1,)), ((), ())),
                         precision=_PREC)
    _wr_partials(x3, p3o_ref)


def _t4_body(xc_ref, nx_ref, w1_ref, w2_ref, w3_ref, g1_ref, b1_ref,
             g2_ref, b2_ref, g3_ref, b3_ref, p1_ref, p2_ref, p3_ref,
             out_ref):
    h1 = _norm_relu(_x1_of(xc_ref, nx_ref, w1_ref), p1_ref, g1_ref, b1_ref)
    x2 = lax.dot_general(h1, w2_ref[...], (((1,), (1,)), ((), ())),
                         precision=_PREC)
    h2 = _norm_relu(x2, p2_ref, g2_ref, b2_ref)
    x3 = lax.dot_general(h2, w3_ref[...], (((1,), (1,)), ((), ())),
                         precision=_PREC)
    mx = jnp.max(jnp.reshape(x3, (MTILE // K, K, C3)), axis=1)
    mu3, iv3 = _stats(p3_ref)
    out_ref[...] = jnp.maximum((mx - mu3) * iv3 * g3_ref[...] + b3_ref[...],
                               0.0)


def _run_mlp(xcat, nxp, W1p, W2, W3, g1, b1, g2, b2, g3, b3):
    g1r, b1r = g1.reshape(1, C1), b1.reshape(1, C1)
    g2r, b2r = g2.reshape(1, C2), b2.reshape(1, C2)
    g3r, b3r = g3.reshape(1, C3), b3.reshape(1, C3)

    xc_spec = pl.BlockSpec((MTILE, CH0), lambda i: (i, 0))
    nx_spec = pl.BlockSpec((MTILE // K, CH0), lambda i: (i, 0))

    def full(a):
        return pl.BlockSpec(a.shape, lambda i: tuple(0 for _ in a.shape))

    p_spec = pl.BlockSpec((1, 2, C1), lambda i: (i, 0, 0))
    p3_spec = pl.BlockSpec((1, 2, C3), lambda i: (i, 0, 0))

    p1 = pl.pallas_call(
        _t1_body, grid=(MGRID,),
        in_specs=[xc_spec, nx_spec, full(W1p)],
        out_specs=p_spec,
        out_shape=jax.ShapeDtypeStruct((MGRID, 2, C1), jnp.float32),
    )(xcat, nxp, W1p)

    p2 = pl.pallas_call(
        _t2_body, grid=(MGRID,),
        in_specs=[xc_spec, nx_spec, full(W1p), full(W2), full(g1r),
                  full(b1r), full(p1)],
        out_specs=p_spec,
        out_shape=jax.ShapeDtypeStruct((MGRID, 2, C2), jnp.float32),
    )(xcat, nxp, W1p, W2, g1r, b1r, p1)

    p3 = pl.pallas_call(
        _t3_body, grid=(MGRID,),
        in_specs=[xc_spec, nx_spec, full(W1p), full(W2), full(W3),
                  full(g1r), full(b1r), full(g2r), full(b2r), full(p1),
                  full(p2)],
        out_specs=p3_spec,
        out_shape=jax.ShapeDtypeStruct((MGRID, 2, C3), jnp.float32),
    )(xcat, nxp, W1p, W2, W3, g1r, b1r, g2r, b2r, p1, p2)

    out = pl.pallas_call(
        _t4_body, grid=(MGRID,),
        in_specs=[xc_spec, nx_spec, full(W1p), full(W2), full(W3),
                  full(g1r), full(b1r), full(g2r), full(b2r), full(g3r),
                  full(b3r), full(p1), full(p2), full(p3)],
        out_specs=pl.BlockSpec((MTILE // K, C3), lambda i: (i, 0)),
        out_shape=jax.ShapeDtypeStruct((ROWS, C3), jnp.float32),
    )(xcat, nxp, W1p, W2, W3, g1r, b1r, g2r, b2r, g3r, b3r, p1, p2, p3)
    return out


def kernel(xyz, features, W1, g1, b1, W2, g2, b2, W3, g3, b3):
    cpad = _run_fps(xyz)
    group = _run_ballquery(xyz, cpad)
    table = jnp.concatenate(
        [xyz, features, jnp.zeros((B, N, CH0 - 35), jnp.float32)],
        axis=-1).reshape(B * N, CH0)
    xcat = _gather_rows(table, group.reshape(XROWS))
    nxp = jnp.concatenate(
        [cpad.reshape(ROWS, 16), jnp.zeros((ROWS, CH0 - 16), jnp.float32)],
        axis=1)
    W1p = jnp.concatenate(
        [W1, jnp.zeros((C1, CH0 - 35), jnp.float32)], axis=1)
    feats = _run_mlp(xcat, nxp, W1p, W2, W3, g1, b1, g2, b2, g3, b3)
    new_xyz = cpad[:, :, 0:3]
    new_features = feats.reshape(B, S, C3)
    return new_xyz, new_features


# speedup vs baseline: 15.8422x; 1.2843x over previous
import functools

import jax
import jax.numpy as jnp
from jax import lax
from jax.experimental import pallas as pl
from jax.experimental.pallas import tpu as pltpu
from jax.experimental.pallas import tpu_sc as plsc

B = 8
N = 8192
S = 1024
K = 32
CH0 = 128
C1, C2, C3 = 32, 32, 64
R2 = 0.2 ** 2
EPSV = 1e-5
ROWS = B * S
XROWS = ROWS * K
TS = 128
MTILE = 2048
MGRID = XROWS // MTILE
NTOT = float(XROWS)

_PREC = lax.Precision.DEFAULT


def _fps_body(xyzT_ref, cpad_ref, dist_ref):
    x = xyzT_ref[0]
    y = xyzT_ref[1]
    z = xyzT_ref[2]
    col = lax.broadcasted_iota(jnp.int32, (B, N), 1)
    dist_ref[...] = jnp.full((B, N), jnp.inf, dtype=jnp.float32)

    def step(t, far):
        fm = col == far
        cx = jnp.sum(jnp.where(fm, x, 0.0), axis=1, keepdims=True)
        cy = jnp.sum(jnp.where(fm, y, 0.0), axis=1, keepdims=True)
        cz = jnp.sum(jnp.where(fm, z, 0.0), axis=1, keepdims=True)
        crow = jnp.concatenate(
            [cx[:, :, None], cy[:, :, None], cz[:, :, None],
             jnp.zeros((B, 1, 13), jnp.float32)], axis=2)
        cpad_ref[:, pl.ds(t, 1), :] = crow
        dx = x - cx
        dy = y - cy
        dz = z - cz
        d = (dx * dx + dy * dy) + dz * dz
        dn = jnp.minimum(dist_ref[...], d)
        dist_ref[...] = dn
        m = jnp.max(dn, axis=1, keepdims=True)
        far_new = jnp.min(jnp.where(dn == m, col, N), axis=1, keepdims=True)
        return far_new

    lax.fori_loop(0, S, step, jnp.zeros((B, 1), jnp.int32))


def _run_fps(xyz):
    xyzT = jnp.transpose(xyz, (2, 0, 1))
    return pl.pallas_call(
        _fps_body,
        out_shape=jax.ShapeDtypeStruct((B, S, 16), jnp.float32),
        scratch_shapes=[pltpu.VMEM((B, N), jnp.float32)],
    )(xyzT)


def _bq_body(xyzT_ref, cpad_ref, out_ref, key_ref):
    p3 = xyzT_ref[0]
    cm = cpad_ref[0][:, 0:3]
    px = p3[0:1, :]
    py = p3[1:2, :]
    pz = p3[2:3, :]
    pn2 = (px * px + py * py) + pz * pz
    cn2 = jnp.sum(cm * cm, axis=1, keepdims=True)
    dmm = lax.dot_general(cm, p3, (((1,), (0,)), ((), ())),
                          precision=_PREC)
    sq = (cn2 + pn2) - 2.0 * dmm
    col = lax.broadcasted_iota(jnp.int32, (TS, N), 1)
    ki = lax.bitcast_convert_type(jnp.maximum(sq, 0.0), jnp.int32)
    packed = (ki & jnp.int32(-8192)) | col
    INF19 = jnp.int32(0x7F800000)
    nearest = jnp.min(packed, axis=1, keepdims=True) & jnp.int32(0x1FFF)
    key_ref[...] = jnp.where(sq <= R2, packed, INF19 | col)
    base = pl.program_id(0) * N
    kcol = lax.broadcasted_iota(jnp.int32, (TS, K), 1)
    DEAD = jnp.int32(0x7FFFFFFF)

    def step(k, acc):
        kv = key_ref[...]
        m = jnp.min(kv, axis=1, keepdims=True)
        sel = jnp.where(m >= INF19, nearest, m & jnp.int32(0x1FFF))
        key_ref[...] = jnp.where(kv == m, DEAD, kv)
        return jnp.where(kcol == k, sel, acc)

    acc = lax.fori_loop(0, K, step, jnp.zeros((TS, K), jnp.int32))
    out_ref[0] = acc + base


def _run_ballquery(xyz, cpad):
    xyzT = jnp.transpose(xyz, (0, 2, 1))
    return pl.pallas_call(
        _bq_body,
        grid=(B, S // TS),
        in_specs=[
            pl.BlockSpec((1, 3, N), lambda b, s: (b, 0, 0)),
            pl.BlockSpec((1, TS, 16), lambda b, s: (b, s, 0)),
        ],
        out_specs=pl.BlockSpec((1, TS, K), lambda b, s: (b, s, 0)),
        out_shape=jax.ShapeDtypeStruct((B, S, K), jnp.int32),
        scratch_shapes=[pltpu.VMEM((TS, N), jnp.int32)],
    )(xyzT, cpad)


_NW = 32
_RPW = XROWS // _NW
_GCHUNK = 256


def _sc_gather_body(table_hbm, gidx_hbm, out_hbm, idx_v, rows_v, sem):
    wid = lax.axis_index("s") * 2 + lax.axis_index("c")
    base = wid * _RPW

    def chunk(g, carry):
        off = base + g * _GCHUNK
        pltpu.sync_copy(gidx_hbm.at[pl.ds(off, _GCHUNK)], idx_v)
        pltpu.async_copy(table_hbm.at[idx_v], rows_v, sem).wait()
        pltpu.sync_copy(rows_v, out_hbm.at[pl.ds(off, _GCHUNK)])
        return carry

    lax.fori_loop(0, _RPW // _GCHUNK, chunk, 0)


def _gather_rows(table, gidx_flat):
    mesh = plsc.VectorSubcoreMesh(core_axis_name="c", subcore_axis_name="s")
    fn = functools.partial(
        pl.kernel,
        out_type=jax.ShapeDtypeStruct((XROWS, CH0), jnp.float32),
        mesh=mesh,
        scratch_types=[
            pltpu.VMEM((_GCHUNK,), jnp.int32),
            pltpu.VMEM((_GCHUNK, CH0), jnp.float32),
            pltpu.SemaphoreType.DMA,
        ],
    )(_sc_gather_body)
    return fn(table, gidx_flat)


def _stats(p_ref):
    ps = p_ref[...]
    mu = jnp.sum(ps[:, 0, :], axis=0, keepdims=True) / NTOT
    ex2 = jnp.sum(ps[:, 1, :], axis=0, keepdims=True) / NTOT
    var = ex2 - mu * mu
    inv = lax.rsqrt(var + EPSV)
    return mu, inv


def _x1_of(xc_ref, nx_ref, w1_ref):
    xc = xc_ref[...]
    cb = nx_ref[...]
    cexp = jnp.reshape(
        jnp.broadcast_to(cb[:, None, :], (MTILE // K, K, CH0)), (MTILE, CH0))
    x0 = xc - cexp
    return lax.dot_general(x0, w1_ref[...], (((1,), (1,)), ((), ())),
                           precision=_PREC)


def _norm_relu(x, p_ref, g_ref, b_ref):
    mu, inv = _stats(p_ref)
    return jnp.maximum((x - mu) * inv * g_ref[...] + b_ref[...], 0.0)


def _wr_partials(x, pout_ref):
    pout_ref[0, 0, :] = jnp.sum(x, axis=0)
    pout_ref[0, 1, :] = jnp.sum(x * x, axis=0)


def _t1_body(xc_ref, nx_ref, w1_ref, p1o_ref):
    _wr_partials(_x1_of(xc_ref, nx_ref, w1_ref), p1o_ref)


def _t2_body(xc_ref, nx_ref, w1_ref, w2_ref, g1_ref, b1_ref, p1_ref, p2o_ref):
    h1 = _norm_relu(_x1_of(xc_ref, nx_ref, w1_ref), p1_ref, g1_ref, b1_ref)
    x2 = lax.dot_general(h1, w2_ref[...], (((1,), (1,)), ((), ())),
                         precision=_PREC)
    _wr_partials(x2, p2o_ref)


def _t3_body(xc_ref, nx_ref, w1_ref, w2_ref, w3_ref, g1_ref, b1_ref,
             g2_ref, b2_ref, p1_ref, p2_ref, p3o_ref):
    h1 = _norm_relu(_x1_of(xc_ref, nx_ref, w1_ref), p1_ref, g1_ref, b1_ref)
    x2 = lax.dot_general(h1, w2_ref[...], (((1,), (1,)), ((), ())),
                         precision=_PREC)
    h2 = _norm_relu(x2, p2_ref, g2_ref, b2_ref)
    x3 = lax.dot_general(h2, w3_ref[...], (((1,), (1,)), ((), ())),
                         precision=_PREC)
    _wr_partials(x3, p3o_ref)


def _t4_body(xc_ref, nx_ref, w1_ref, w2_ref, w3_ref, g1_ref, b1_ref,
             g2_ref, b2_ref, g3_ref, b3_ref, p1_ref, p2_ref, p3_ref,
             out_ref):
    h1 = _norm_relu(_x1_of(xc_ref, nx_ref, w1_ref), p1_ref, g1_ref, b1_ref)
    x2 = lax.dot_general(h1, w2_ref[...], (((1,), (1,)), ((), ())),
                         precision=_PREC)
    h2 = _norm_relu(x2, p2_ref, g2_ref, b2_ref)
    x3 = lax.dot_general(h2, w3_ref[...], (((1,), (1,)), ((), ())),
                         precision=_PREC)
    mx = jnp.max(jnp.reshape(x3, (MTILE // K, K, C3)), axis=1)
    mu3, iv3 = _stats(p3_ref)
    out_ref[...] = jnp.maximum((mx - mu3) * iv3 * g3_ref[...] + b3_ref[...],
                               0.0)


def _run_mlp(xcat, nxp, W1p, W2, W3, g1, b1, g2, b2, g3, b3):
    g1r, b1r = g1.reshape(1, C1), b1.reshape(1, C1)
    g2r, b2r = g2.reshape(1, C2), b2.reshape(1, C2)
    g3r, b3r = g3.reshape(1, C3), b3.reshape(1, C3)

    xc_spec = pl.BlockSpec((MTILE, CH0), lambda i: (i, 0))
    nx_spec = pl.BlockSpec((MTILE // K, CH0), lambda i: (i, 0))

    def full(a):
        return pl.BlockSpec(a.shape, lambda i: tuple(0 for _ in a.shape))

    p_spec = pl.BlockSpec((1, 2, C1), lambda i: (i, 0, 0))
    p3_spec = pl.BlockSpec((1, 2, C3), lambda i: (i, 0, 0))

    p1 = pl.pallas_call(
        _t1_body, grid=(MGRID,),
        in_specs=[xc_spec, nx_spec, full(W1p)],
        out_specs=p_spec,
        out_shape=jax.ShapeDtypeStruct((MGRID, 2, C1), jnp.float32),
    )(xcat, nxp, W1p)

    p2 = pl.pallas_call(
        _t2_body, grid=(MGRID,),
        in_specs=[xc_spec, nx_spec, full(W1p), full(W2), full(g1r),
                  full(b1r), full(p1)],
        out_specs=p_spec,
        out_shape=jax.ShapeDtypeStruct((MGRID, 2, C2), jnp.float32),
    )(xcat, nxp, W1p, W2, g1r, b1r, p1)

    p3 = pl.pallas_call(
        _t3_body, grid=(MGRID,),
        in_specs=[xc_spec, nx_spec, full(W1p), full(W2), full(W3),
                  full(g1r), full(b1r), full(g2r), full(b2r), full(p1),
                  full(p2)],
        out_specs=p3_spec,
        out_shape=jax.ShapeDtypeStruct((MGRID, 2, C3), jnp.float32),
    )(xcat, nxp, W1p, W2, W3, g1r, b1r, g2r, b2r, p1, p2)

    out = pl.pallas_call(
        _t4_body, grid=(MGRID,),
        in_specs=[xc_spec, nx_spec, full(W1p), full(W2), full(W3),
                  full(g1r), full(b1r), full(g2r), full(b2r), full(g3r),
                  full(b3r), full(p1), full(p2), full(p3)],
        out_specs=pl.BlockSpec((MTILE // K, C3), lambda i: (i, 0)),
        out_shape=jax.ShapeDtypeStruct((ROWS, C3), jnp.float32),
    )(xcat, nxp, W1p, W2, W3, g1r, b1r, g2r, b2r, g3r, b3r, p1, p2, p3)
    return out


def kernel(xyz, features, W1, g1, b1, W2, g2, b2, W3, g3, b3):
    cpad = _run_fps(xyz)
    group = _run_ballquery(xyz, cpad)
    table = jnp.concatenate(
        [xyz, features, jnp.zeros((B, N, CH0 - 35), jnp.float32)],
        axis=-1).reshape(B * N, CH0)
    xcat = _gather_rows(table, group.reshape(XROWS))
    nxp = jnp.concatenate(
        [cpad.reshape(ROWS, 16), jnp.zeros((ROWS, CH0 - 16), jnp.float32)],
        axis=1)
    W1p = jnp.concatenate(
        [W1, jnp.zeros((C1, CH0 - 35), jnp.float32)], axis=1)
    feats = _run_mlp(xcat, nxp, W1p, W2, W3, g1, b1, g2, b2, g3, b3)
    new_xyz = cpad[:, :, 0:3]
    new_features = feats.reshape(B, S, C3)
    return new_xyz, new_features
